# baseline (speedup 1.0000x reference)
import jax
import jax.numpy as jnp
from jax.experimental import pallas as pl

_NU = 50000
_NI = 50000
_NB = 20000
_L = 2


def _l2n(x):
    n = jnp.linalg.norm(x, axis=1, keepdims=True)
    return x / jnp.maximum(n, 1e-12)


def _sym(edge_index, n_a, n_b):
    row = edge_index[0]
    col = edge_index[1] + n_a
    rows = jnp.concatenate([row, col])
    cols = jnp.concatenate([col, row])
    N = n_a + n_b
    deg = jax.ops.segment_sum(jnp.ones_like(rows, dtype=jnp.float32), rows, num_segments=N)
    inv = 1.0 / (jnp.sqrt(deg) + 1e-8)
    vals = inv[rows] * inv[cols]
    return rows, cols, vals, N


def _prop(rows, cols, vals, N, a, b):
    f = jnp.concatenate([a, b], axis=0)
    allf = [f]
    for i in range(_L):
        f = jax.ops.segment_sum(vals[:, None] * f[cols], rows, num_segments=N)
        f = f / (i + 2)
        allf.append(_l2n(f))
    tot = jnp.sum(jnp.stack(allf, axis=1), axis=1)
    return tot[: a.shape[0]], tot[a.shape[0]:]


def _copy_body(x_ref, o_ref):
    o_ref[...] = x_ref[...]


def kernel(users_feature, bundles_feature, items_feature, ui_edge_index, ub_edge_index, bi_edge_index):
    r1, c1, v1, N1 = _sym(ui_edge_index, _NU, _NI)
    ILu, ILi = _prop(r1, c1, v1, N1, users_feature, items_feature)
    r2, c2, v2, N2 = _sym(ub_edge_index, _NU, _NB)
    BLu, BLb = _prop(r2, c2, v2, N2, users_feature, bundles_feature)
    brow = bi_edge_index[0]
    bcol = bi_edge_index[1]
    bs = jax.ops.segment_sum(jnp.ones_like(brow, dtype=jnp.float32), brow, num_segments=_NB) + 1e-8
    bvals = (1.0 / bs)[brow]
    ILb = jax.ops.segment_sum(bvals[:, None] * ILi[bcol], brow, num_segments=_NB)
    urep = jnp.concatenate([ILu, BLu], axis=1)
    brep = jnp.concatenate([ILb, BLb], axis=1)
    out = jnp.concatenate([urep, brep], axis=0)
    out = pl.pallas_call(
        _copy_body,
        grid=(70,),
        in_specs=[pl.BlockSpec((1000, 128), lambda i: (i, 0))],
        out_specs=pl.BlockSpec((1000, 128), lambda i: (i, 0)),
        out_shape=jax.ShapeDtypeStruct(out.shape, out.dtype),
    )(out)
    return out


# speedup vs baseline: 8.9666x; 8.9666x over previous
import functools

import jax
import jax.numpy as jnp
from jax import lax
from jax.experimental import pallas as pl
from jax.experimental.pallas import tpu as pltpu
from jax.experimental.pallas import tpu_sc as plsc

_NU = 50000
_NI = 50000
_NB = 20000
_D = 64
_K = 128
_CH = 40
_NCORES = 2
_NSUB = 16


def _ceil_to(x, m):
    return ((x + m - 1) // m) * m


def _mesh():
    return plsc.VectorSubcoreMesh(core_axis_name="c", subcore_axis_name="s")


def _make_deg(E, N):
    nchunks = E // _K
    assert nchunks * _K == E and nchunks % 2 == 0
    R = _ceil_to(N, _NSUB * _CH)
    per_core = nchunks // 2

    @functools.partial(
        pl.kernel,
        mesh=_mesh(),
        out_type=jax.ShapeDtypeStruct((2, R, 16), jnp.float32),
        compiler_params=pltpu.CompilerParams(use_tc_tiling_on_sc=False),
        scratch_types=[
            pltpu.VMEM((_K,), jnp.int32),
            pltpu.VMEM((_K, 16), jnp.float32),
            pltpu.VMEM((_CH, 16), jnp.float32),
            pltpu.VMEM_SHARED((R, 16), jnp.float32),
        ],
    )
    def deg(idx_hbm, out_hbm, idxv, ones, zbuf, acc):
        c = lax.axis_index("c")
        s = lax.axis_index("s")

        def fill(r, t):
            ones[r] = jnp.ones((16,), jnp.float32)
            return t

        lax.fori_loop(0, _K, fill, 0)

        def zfill(r, t):
            zbuf[r] = jnp.zeros((16,), jnp.float32)
            return t

        lax.fori_loop(0, _CH, zfill, 0)

        nz = R // (_NSUB * _CH)

        def zero(m, t):
            base = (s * nz + m) * _CH
            pltpu.sync_copy(zbuf, acc.at[pl.ds(base, _CH)])
            return t

        lax.fori_loop(0, nz, zero, 0)
        plsc.subcore_barrier()

        nci = (per_core - s + _NSUB - 1) // _NSUB

        def body(m, t):
            i = 2 * (m * _NSUB + s) + c
            pltpu.sync_copy(idx_hbm.at[pl.ds(i * _K, _K)], idxv)
            pltpu.sync_copy(ones, acc.at[idxv], add=True)
            return t

        lax.fori_loop(0, nci, body, 0)
        plsc.subcore_barrier()

        def drain(m, t):
            base = (s * nz + m) * _CH
            pltpu.sync_copy(acc.at[pl.ds(base, _CH)], zbuf)
            pltpu.sync_copy(zbuf, out_hbm.at[c, pl.ds(base, _CH)])
            return t

        lax.fori_loop(0, nz, drain, 0)

    return deg


def _make_spmm(E, n_dst):
    nchunks = E // _K
    assert nchunks * _K == E
    assert n_dst % 2 == 0
    P = n_dst // 2
    assert P % _CH == 0
    R = _ceil_to(P + 8, _NSUB * _CH)

    @functools.partial(
        pl.kernel,
        mesh=_mesh(),
        out_type=jax.ShapeDtypeStruct((n_dst, _D), jnp.float32),
        compiler_params=pltpu.CompilerParams(use_tc_tiling_on_sc=False),
        scratch_types=[
            pltpu.VMEM((_K,), jnp.int32),
            pltpu.VMEM((_K,), jnp.int32),
            pltpu.VMEM((_K,), jnp.int32),
            pltpu.VMEM((_K, _D), jnp.float32),
            pltpu.VMEM_SHARED((R, _D), jnp.float32),
            pltpu.SemaphoreType.DMA,
        ],
    )
    def spmm(dst_hbm, src_hbm, tbl_hbm, out_hbm, dstv, srcv, locv, rows, acc, sem):
        c = lax.axis_index("c")
        s = lax.axis_index("s")

        def zfill(r, t):
            for j in range(_D // 16):
                rows[r, pl.ds(j * 16, 16)] = jnp.zeros((16,), jnp.float32)
            return t

        lax.fori_loop(0, _CH, zfill, 0)
        nz = R // (_NSUB * _CH)

        def zero(m, t):
            base = (s * nz + m) * _CH
            pltpu.sync_copy(rows.at[pl.ds(0, _CH)], acc.at[pl.ds(base, _CH)])
            return t

        lax.fori_loop(0, nz, zero, 0)
        plsc.subcore_barrier()

        base_dst = c * P
        nci = (nchunks - s + _NSUB - 1) // _NSUB

        def body(m, t):
            i = m * _NSUB + s
            pltpu.sync_copy(dst_hbm.at[pl.ds(i * _K, _K)], dstv)
            pltpu.sync_copy(src_hbm.at[pl.ds(i * _K, _K)], srcv)
            cp = pltpu.async_copy(tbl_hbm.at[srcv], rows, sem)
            for j in range(_K // 16):
                d = dstv[pl.ds(j * 16, 16)]
                loc = d - base_dst
                ok = (loc >= 0) & (loc < P)
                locv[pl.ds(j * 16, 16)] = jnp.where(ok, loc, P)
            cp.wait()
            pltpu.sync_copy(rows, acc.at[locv], add=True)
            return t

        lax.fori_loop(0, nci, body, 0)
        plsc.subcore_barrier()

        nk = P // _CH
        nkt = (nk - s + _NSUB - 1) // _NSUB

        def drain(m, t):
            k = m * _NSUB + s
            pltpu.sync_copy(acc.at[pl.ds(k * _CH, _CH)], rows.at[pl.ds(0, _CH)])
            pltpu.sync_copy(rows.at[pl.ds(0, _CH)], out_hbm.at[pl.ds(base_dst + k * _CH, _CH)])
            return t

        lax.fori_loop(0, nkt, drain, 0)

    return spmm


_B = 1000


def _inv_of(d_ref):
    deg = d_ref[0] + d_ref[1]
    return 1.0 / (jnp.sqrt(deg) + 1e-8)


def _scale_body(x_ref, d_ref, o_ref):
    o_ref[...] = x_ref[...] * _inv_of(d_ref)


def _mid_body(s_ref, d_ref, z_ref, y_ref, *, denom):
    inv = _inv_of(d_ref)
    f = s_ref[...] * inv * (1.0 / denom)
    nrm = jnp.sqrt(jnp.sum(f * f, axis=1, keepdims=True))
    z_ref[...] = f / jnp.maximum(nrm, 1e-12)
    y_ref[...] = f * inv


def _fin_body(s_ref, d_ref, x_ref, z1_ref, o_ref, *, denom):
    inv = _inv_of(d_ref)
    f = s_ref[...] * inv * (1.0 / denom)
    nrm = jnp.sqrt(jnp.sum(f * f, axis=1, keepdims=True))
    z2 = f / jnp.maximum(nrm, 1e-12)
    o_ref[...] = x_ref[...] + z1_ref[...] + z2


def _bi_body(s_ref, d_ref, o_ref):
    cnt = d_ref[0] + d_ref[1]
    o_ref[...] = s_ref[...] * (1.0 / (cnt + 1e-8))


def _row_spec(n):
    return pl.BlockSpec((_B, _D), lambda i: (i, 0))


def _deg_spec():
    return pl.BlockSpec((2, _B, 1), lambda i: (0, i, 0))


def _tc_call(body, n, n_out, *args):
    outs = [jax.ShapeDtypeStruct((n, _D), jnp.float32)] * n_out
    in_specs = []
    for a in args:
        if a.ndim == 3:
            in_specs.append(_deg_spec())
        else:
            in_specs.append(_row_spec(n))
    res = pl.pallas_call(
        body,
        grid=(n // _B,),
        in_specs=in_specs,
        out_specs=[_row_spec(n)] * n_out,
        out_shape=outs,
    )(*args)
    return res


def _propagate(a_feat, b_feat, edge, n_a, n_b):
    E = edge.shape[1]
    e0 = edge[0].astype(jnp.int32)
    e1 = edge[1].astype(jnp.int32)
    N = n_a + n_b
    all_idx = jnp.concatenate([e0, e1 + n_a])
    degp = _make_deg(2 * E, N)(all_idx)
    da = degp[:, :n_a, :1]
    db = degp[:, n_a:N, :1]

    ya0 = _tc_call(_scale_body, n_a, 1, a_feat, da)[0]
    yb0 = _tc_call(_scale_body, n_b, 1, b_feat, db)[0]

    spmm_a = _make_spmm(E, n_a)
    spmm_b = _make_spmm(E, n_b)

    s1a = spmm_a(e0, e1, yb0)
    s1b = spmm_b(e1, e0, ya0)
    z1a, y1a = _tc_call(functools.partial(_mid_body, denom=2.0), n_a, 2, s1a, da)
    z1b, y1b = _tc_call(functools.partial(_mid_body, denom=2.0), n_b, 2, s1b, db)

    s2a = spmm_a(e0, e1, y1b)
    s2b = spmm_b(e1, e0, y1a)
    tot_a = _tc_call(functools.partial(_fin_body, denom=3.0), n_a, 1, s2a, da, a_feat, z1a)[0]
    tot_b = _tc_call(functools.partial(_fin_body, denom=3.0), n_b, 1, s2b, db, b_feat, z1b)[0]
    return tot_a, tot_b


def kernel(users_feature, bundles_feature, items_feature, ui_edge_index, ub_edge_index, bi_edge_index):
    IL_users, IL_items = _propagate(users_feature, items_feature, ui_edge_index, _NU, _NI)
    BL_users, BL_bundles = _propagate(users_feature, bundles_feature, ub_edge_index, _NU, _NB)

    brow = bi_edge_index[0].astype(jnp.int32)
    bcol = bi_edge_index[1].astype(jnp.int32)
    Eb = brow.shape[0]
    sizep = _make_deg(Eb, _NB)(brow)
    dbi = sizep[:, :_NB, :1]
    sbi = _make_spmm(Eb, _NB)(brow, bcol, IL_items)
    IL_bundles = _tc_call(_bi_body, _NB, 1, sbi, dbi)[0]

    users_rep = jnp.concatenate([IL_users, BL_users], axis=1)
    bundles_rep = jnp.concatenate([IL_bundles, BL_bundles], axis=1)
    return jnp.concatenate([users_rep, bundles_rep], axis=0)


# speedup vs baseline: 14.9849x; 1.6712x over previous
import functools

import jax
import jax.numpy as jnp
from jax import lax
from jax.experimental import pallas as pl
from jax.experimental.pallas import tpu as pltpu
from jax.experimental.pallas import tpu_sc as plsc

_NU = 50000
_NI = 50000
_NB = 20000
_D = 64
_H = _D // 2
_K = 128
_CH = 40
_NSUB = 16


def _ceil_to(x, m):
    return ((x + m - 1) // m) * m


def _mesh():
    return plsc.VectorSubcoreMesh(core_axis_name="c", subcore_axis_name="s")


def _make_deg(E, N):
    nchunks = E // _K
    assert nchunks * _K == E and nchunks % 2 == 0
    R = _ceil_to(N, _NSUB * _CH)
    per_core = nchunks // 2

    @functools.partial(
        pl.kernel,
        mesh=_mesh(),
        out_type=jax.ShapeDtypeStruct((2, R, 16), jnp.float32),
        compiler_params=pltpu.CompilerParams(use_tc_tiling_on_sc=False),
        scratch_types=[
            pltpu.VMEM((_K,), jnp.int32),
            pltpu.VMEM((_K, 16), jnp.float32),
            pltpu.VMEM((_CH, 16), jnp.float32),
            pltpu.VMEM_SHARED((R, 16), jnp.float32),
        ],
    )
    def deg(idx_hbm, out_hbm, idxv, ones, zbuf, acc):
        c = lax.axis_index("c")
        s = lax.axis_index("s")

        def fill(r, t):
            ones[r] = jnp.ones((16,), jnp.float32)
            return t

        lax.fori_loop(0, _K, fill, 0)

        def zfill(r, t):
            zbuf[r] = jnp.zeros((16,), jnp.float32)
            return t

        lax.fori_loop(0, _CH, zfill, 0)

        nz = R // (_NSUB * _CH)

        def zero(m, t):
            base = (s * nz + m) * _CH
            pltpu.sync_copy(zbuf, acc.at[pl.ds(base, _CH)])
            return t

        lax.fori_loop(0, nz, zero, 0)
        plsc.subcore_barrier()

        nci = (per_core - s + _NSUB - 1) // _NSUB

        def body(m, t):
            i = 2 * (m * _NSUB + s) + c
            pltpu.sync_copy(idx_hbm.at[pl.ds(i * _K, _K)], idxv)
            pltpu.sync_copy(ones, acc.at[idxv], add=True)
            return t

        lax.fori_loop(0, nci, body, 0)
        plsc.subcore_barrier()

        def drain(m, t):
            base = (s * nz + m) * _CH
            pltpu.sync_copy(acc.at[pl.ds(base, _CH)], zbuf)
            pltpu.sync_copy(zbuf, out_hbm.at[c, pl.ds(base, _CH)])
            return t

        lax.fori_loop(0, nz, drain, 0)

    return deg


def _make_spmm(E, n_dst, V):
    nchunks = E // _K
    assert nchunks * _K == E
    assert n_dst % _CH == 0
    R = _ceil_to(n_dst, _NSUB * _CH)

    @functools.partial(
        pl.kernel,
        mesh=_mesh(),
        out_type=jax.ShapeDtypeStruct((2, n_dst, _H), jnp.float32),
        compiler_params=pltpu.CompilerParams(use_tc_tiling_on_sc=False),
        scratch_types=[
            pltpu.VMEM((_K,), jnp.int32),
            pltpu.VMEM((_K,), jnp.int32),
            pltpu.VMEM((_K,), jnp.int32),
            pltpu.VMEM((_K,), jnp.int32),
            pltpu.VMEM((_K, _H), jnp.float32),
            pltpu.VMEM((_K, _H), jnp.float32),
            pltpu.VMEM_SHARED((R, _H), jnp.float32),
            pltpu.SemaphoreType.DMA,
            pltpu.SemaphoreType.DMA,
        ],
    )
    def spmm(dst_hbm, src_hbm, tbl_hbm, out_hbm,
             src0, src1, dst0, dst1, rows0, rows1, acc, sem0, sem1):
        c = lax.axis_index("c")
        s = lax.axis_index("s")
        off = c * V

        def zfill(r, t):
            for j in range(_H // 16):
                rows0[r, pl.ds(j * 16, 16)] = jnp.zeros((16,), jnp.float32)
            return t

        lax.fori_loop(0, _CH, zfill, 0)
        nz = R // (_NSUB * _CH)

        def zero(m, t):
            base = (s * nz + m) * _CH
            pltpu.sync_copy(rows0.at[pl.ds(0, _CH)], acc.at[pl.ds(base, _CH)])
            return t

        lax.fori_loop(0, nz, zero, 0)
        plsc.subcore_barrier()

        nci = (nchunks - s + _NSUB - 1) // _NSUB

        def issue(k, srcb, dstb, rowsb, sem):
            base = (k * _NSUB + s) * _K
            pltpu.sync_copy(dst_hbm.at[pl.ds(base, _K)], dstb)
            pltpu.sync_copy(src_hbm.at[pl.ds(base, _K)], srcb)
            for j in range(_K // 16):
                sl = pl.ds(j * 16, 16)
                srcb[sl] = srcb[sl] + off
            pltpu.make_async_copy(tbl_hbm.at[srcb], rowsb, sem).start()

        def fire(k, srcb, dstb, rowsb, sem):
            @pl.when(k < nci)
            def _():
                issue(k, srcb, dstb, rowsb, sem)

        def drain_chunk(srcb, dstb, rowsb, sem):
            pltpu.make_async_copy(tbl_hbm.at[srcb], rowsb, sem).wait()
            pltpu.sync_copy(rowsb, acc.at[dstb], add=True)

        issue(0, src0, dst0, rows0, sem0)

        def body(mm, t):
            k0 = 2 * mm
            k1 = k0 + 1

            @pl.when(k0 < nci)
            def _():
                fire(k1, src1, dst1, rows1, sem1)
                drain_chunk(src0, dst0, rows0, sem0)

            @pl.when(k1 < nci)
            def _():
                fire(k1 + 1, src0, dst0, rows0, sem0)
                drain_chunk(src1, dst1, rows1, sem1)

            return t

        lax.fori_loop(0, (nci + 1) // 2, body, 0)
        plsc.subcore_barrier()

        nk = n_dst // _CH
        nkt = (nk - s + _NSUB - 1) // _NSUB

        def drain(m, t):
            k = m * _NSUB + s
            pltpu.sync_copy(acc.at[pl.ds(k * _CH, _CH)], rows0.at[pl.ds(0, _CH)])
            pltpu.sync_copy(rows0.at[pl.ds(0, _CH)], out_hbm.at[c, pl.ds(k * _CH, _CH)])
            return t

        lax.fori_loop(0, nkt, drain, 0)

    return spmm


_B = 1000


def _inv_of(d_ref):
    deg = d_ref[0] + d_ref[1]
    return 1.0 / (jnp.sqrt(deg) + 1e-8)


def _scale_body(x_ref, d_ref, y_ref):
    inv = _inv_of(d_ref)
    y_ref[0] = x_ref[:, :_H] * inv
    y_ref[1] = x_ref[:, _H:] * inv


def _mid_body(s_ref, d_ref, z_ref, y_ref, *, denom):
    inv = _inv_of(d_ref)
    f0 = s_ref[0] * inv * (1.0 / denom)
    f1 = s_ref[1] * inv * (1.0 / denom)
    nrm2 = jnp.sum(f0 * f0, axis=1, keepdims=True) + jnp.sum(f1 * f1, axis=1, keepdims=True)
    scale = 1.0 / jnp.maximum(jnp.sqrt(nrm2), 1e-12)
    z_ref[...] = jnp.concatenate([f0, f1], axis=1) * scale
    y_ref[0] = f0 * inv
    y_ref[1] = f1 * inv


def _fin_body(s_ref, d_ref, x_ref, z1_ref, o_ref, *, denom):
    inv = _inv_of(d_ref)
    f0 = s_ref[0] * inv * (1.0 / denom)
    f1 = s_ref[1] * inv * (1.0 / denom)
    nrm2 = jnp.sum(f0 * f0, axis=1, keepdims=True) + jnp.sum(f1 * f1, axis=1, keepdims=True)
    scale = 1.0 / jnp.maximum(jnp.sqrt(nrm2), 1e-12)
    z2 = jnp.concatenate([f0, f1], axis=1) * scale
    o_ref[...] = x_ref[...] + z1_ref[...] + z2


def _fin_split_body(s_ref, d_ref, x_ref, z1_ref, o_ref, o2_ref, *, denom):
    inv = _inv_of(d_ref)
    f0 = s_ref[0] * inv * (1.0 / denom)
    f1 = s_ref[1] * inv * (1.0 / denom)
    nrm2 = jnp.sum(f0 * f0, axis=1, keepdims=True) + jnp.sum(f1 * f1, axis=1, keepdims=True)
    scale = 1.0 / jnp.maximum(jnp.sqrt(nrm2), 1e-12)
    z2 = jnp.concatenate([f0, f1], axis=1) * scale
    o = x_ref[...] + z1_ref[...] + z2
    o_ref[...] = o
    o2_ref[0] = o[:, :_H]
    o2_ref[1] = o[:, _H:]


def _bi_body(s_ref, d_ref, o_ref):
    cnt = d_ref[0] + d_ref[1]
    scale = 1.0 / (cnt + 1e-8)
    o_ref[...] = jnp.concatenate([s_ref[0], s_ref[1]], axis=1) * scale


def _spec_of(a):
    if a.ndim == 3 and a.shape[2] == 1:
        return pl.BlockSpec((2, _B, 1), lambda i: (0, i, 0))
    if a.ndim == 3:
        return pl.BlockSpec((2, _B, _H), lambda i: (0, i, 0))
    return pl.BlockSpec((_B, _D), lambda i: (i, 0))


def _tc_call(body, n, out_kinds, *args):
    shapes = {"full": jax.ShapeDtypeStruct((n, _D), jnp.float32),
              "split": jax.ShapeDtypeStruct((2, n, _H), jnp.float32)}
    specs = {"full": pl.BlockSpec((_B, _D), lambda i: (i, 0)),
             "split": pl.BlockSpec((2, _B, _H), lambda i: (0, i, 0))}
    res = pl.pallas_call(
        body,
        grid=(n // _B,),
        in_specs=[_spec_of(a) for a in args],
        out_specs=[specs[k] for k in out_kinds],
        out_shape=[shapes[k] for k in out_kinds],
    )(*args)
    return res


def _propagate(a_feat, b_feat, edge, n_a, n_b, split_tot_b=False):
    E = edge.shape[1]
    e0 = edge[0].astype(jnp.int32)
    e1 = edge[1].astype(jnp.int32)
    N = n_a + n_b
    all_idx = jnp.concatenate([e0, e1 + n_a])
    degp = _make_deg(2 * E, N)(all_idx)
    da = degp[:, :n_a, :1]
    db = degp[:, n_a:N, :1]

    ya0 = _tc_call(_scale_body, n_a, ["split"], a_feat, da)[0]
    yb0 = _tc_call(_scale_body, n_b, ["split"], b_feat, db)[0]

    spmm_a = _make_spmm(E, n_a, n_b)
    spmm_b = _make_spmm(E, n_b, n_a)

    s1a = spmm_a(e0, e1, yb0.reshape(2 * n_b, _H))
    s1b = spmm_b(e1, e0, ya0.reshape(2 * n_a, _H))
    z1a, y1a = _tc_call(functools.partial(_mid_body, denom=2.0), n_a, ["full", "split"], s1a, da)
    z1b, y1b = _tc_call(functools.partial(_mid_body, denom=2.0), n_b, ["full", "split"], s1b, db)

    s2a = spmm_a(e0, e1, y1b.reshape(2 * n_b, _H))
    s2b = spmm_b(e1, e0, y1a.reshape(2 * n_a, _H))
    tot_a = _tc_call(functools.partial(_fin_body, denom=3.0), n_a, ["full"], s2a, da, a_feat, z1a)[0]
    if split_tot_b:
        tot_b, tot_b_split = _tc_call(
            functools.partial(_fin_split_body, denom=3.0), n_b, ["full", "split"], s2b, db, b_feat, z1b)
        return tot_a, tot_b, tot_b_split
    tot_b = _tc_call(functools.partial(_fin_body, denom=3.0), n_b, ["full"], s2b, db, b_feat, z1b)[0]
    return tot_a, tot_b


def kernel(users_feature, bundles_feature, items_feature, ui_edge_index, ub_edge_index, bi_edge_index):
    IL_users, IL_items, IL_items_split = _propagate(
        users_feature, items_feature, ui_edge_index, _NU, _NI, split_tot_b=True)
    BL_users, BL_bundles = _propagate(users_feature, bundles_feature, ub_edge_index, _NU, _NB)

    brow = bi_edge_index[0].astype(jnp.int32)
    bcol = bi_edge_index[1].astype(jnp.int32)
    Eb = brow.shape[0]
    sizep = _make_deg(Eb, _NB)(brow)
    dbi = sizep[:, :_NB, :1]
    sbi = _make_spmm(Eb, _NB, _NI)(brow, bcol, IL_items_split.reshape(2 * _NI, _H))
    IL_bundles = _tc_call(_bi_body, _NB, ["full"], sbi, dbi)[0]

    users_rep = jnp.concatenate([IL_users, BL_users], axis=1)
    bundles_rep = jnp.concatenate([IL_bundles, BL_bundles], axis=1)
    return jnp.concatenate([users_rep, bundles_rep], axis=0)


# speedup vs baseline: 22.4197x; 1.4962x over previous
import functools

import jax
import jax.numpy as jnp
from jax import lax
from jax.experimental import pallas as pl
from jax.experimental.pallas import tpu as pltpu
from jax.experimental.pallas import tpu_sc as plsc

_NU = 50000
_NI = 50000
_NB = 20000
_D = 64
_H = _D // 2
_K = 128
_CH = 40
_NSUB = 16


def _ceil_to(x, m):
    return ((x + m - 1) // m) * m


def _mesh():
    return plsc.VectorSubcoreMesh(core_axis_name="c", subcore_axis_name="s")


_SC4 = 4


def _make_deg(E, N):
    nchunks = E // _K
    assert nchunks * _K == E and nchunks % _SC4 == 0
    nsuper = nchunks // _SC4
    R = _ceil_to(N, _NSUB * _CH)

    @functools.partial(
        pl.kernel,
        mesh=_mesh(),
        out_type=jax.ShapeDtypeStruct((2, R, 16), jnp.float32),
        compiler_params=pltpu.CompilerParams(use_tc_tiling_on_sc=False),
        scratch_types=[
            pltpu.VMEM((_SC4, _K), jnp.int32),
            pltpu.VMEM((_K, 16), jnp.float32),
            pltpu.VMEM((_CH, 16), jnp.float32),
            pltpu.VMEM_SHARED((R, 16), jnp.float32),
            pltpu.SemaphoreType.DMA,
        ],
    )
    def deg(idx_hbm, out_hbm, idxv, ones, zbuf, acc, sem):
        c = lax.axis_index("c")
        s = lax.axis_index("s")
        wid = s * 2 + c

        def fill(r, t):
            ones[r] = jnp.ones((16,), jnp.float32)
            return t

        lax.fori_loop(0, _K, fill, 0)

        def zfill(r, t):
            zbuf[r] = jnp.zeros((16,), jnp.float32)
            return t

        lax.fori_loop(0, _CH, zfill, 0)

        nz = R // (_NSUB * _CH)

        def zero(m, t):
            base = (s * nz + m) * _CH
            pltpu.sync_copy(zbuf, acc.at[pl.ds(base, _CH)])
            return t

        lax.fori_loop(0, nz, zero, 0)
        plsc.subcore_barrier()

        nci = (nsuper - wid + 31) // 32

        def body(m, t):
            u = m * 32 + wid
            pltpu.sync_copy(idx_hbm.at[pl.ds(u * _SC4, _SC4)], idxv)
            for j in range(_SC4):
                pltpu.make_async_copy(ones, acc.at[idxv.at[j]], sem).start(add=True)
            for j in range(_SC4):
                pltpu.make_async_copy(ones, acc.at[idxv.at[j]], sem).wait()
            return t

        lax.fori_loop(0, nci, body, 0)
        plsc.subcore_barrier()

        def drain(m, t):
            base = (s * nz + m) * _CH
            pltpu.sync_copy(acc.at[pl.ds(base, _CH)], zbuf)
            pltpu.sync_copy(zbuf, out_hbm.at[c, pl.ds(base, _CH)])
            return t

        lax.fori_loop(0, nz, drain, 0)

    return deg


_SB = 2


def _make_spmm(E, n_dst, V):
    nchunks = E // _K
    assert nchunks * _K == E and nchunks % _SB == 0
    nsuper = nchunks // _SB
    assert n_dst % _CH == 0
    R = _ceil_to(n_dst, _NSUB * _CH)

    @functools.partial(
        pl.kernel,
        mesh=_mesh(),
        out_type=jax.ShapeDtypeStruct((2, n_dst, _H), jnp.float32),
        compiler_params=pltpu.CompilerParams(use_tc_tiling_on_sc=False),
        scratch_types=[
            pltpu.VMEM((_SB, _K), jnp.int32),
            pltpu.VMEM((_SB, _K), jnp.int32),
            pltpu.VMEM((_SB, _K), jnp.int32),
            pltpu.VMEM((_SB, _K), jnp.int32),
            pltpu.VMEM((_SB, _K, _H), jnp.float32),
            pltpu.VMEM((_SB, _K, _H), jnp.float32),
            pltpu.VMEM_SHARED((R, _H), jnp.float32),
            pltpu.SemaphoreType.DMA,
            pltpu.SemaphoreType.DMA,
        ],
    )
    def spmm(dst_hbm, src_hbm, tbl_hbm, out_hbm,
             src0, src1, dst0, dst1, rows0, rows1, acc, sem0, sem1):
        c = lax.axis_index("c")
        s = lax.axis_index("s")
        off = c * V

        def zfill(r, t):
            for j in range(_H // 16):
                rows0[0, r, pl.ds(j * 16, 16)] = jnp.zeros((16,), jnp.float32)
            return t

        lax.fori_loop(0, _CH, zfill, 0)
        nz = R // (_NSUB * _CH)

        def zero(m, t):
            base = (s * nz + m) * _CH
            pltpu.sync_copy(rows0.at[0, pl.ds(0, _CH)], acc.at[pl.ds(base, _CH)])
            return t

        lax.fori_loop(0, nz, zero, 0)
        plsc.subcore_barrier()

        nci = (nsuper - s + _NSUB - 1) // _NSUB

        def issue(k, srcb, dstb, rowsb, sem):
            row = (k * _NSUB + s) * _SB
            pltpu.sync_copy(dst_hbm.at[pl.ds(row, _SB)], dstb)
            pltpu.sync_copy(src_hbm.at[pl.ds(row, _SB)], srcb)
            for j2 in range(_SB):
                for j in range(_K // 16):
                    sl = pl.ds(j * 16, 16)
                    srcb[j2, sl] = srcb[j2, sl] + off
            for j2 in range(_SB):
                pltpu.make_async_copy(tbl_hbm.at[srcb.at[j2]], rowsb.at[j2], sem).start()

        def fire(k, srcb, dstb, rowsb, sem):
            @pl.when(k < nci)
            def _():
                issue(k, srcb, dstb, rowsb, sem)

        def drain_chunk(srcb, dstb, rowsb, sem):
            for j2 in range(_SB):
                pltpu.make_async_copy(tbl_hbm.at[srcb.at[j2]], rowsb.at[j2], sem).wait()
            for j2 in range(_SB):
                pltpu.sync_copy(rowsb.at[j2], acc.at[dstb.at[j2]], add=True)

        issue(0, src0, dst0, rows0, sem0)

        def body(mm, t):
            k0 = 2 * mm
            k1 = k0 + 1

            @pl.when(k0 < nci)
            def _():
                fire(k1, src1, dst1, rows1, sem1)
                drain_chunk(src0, dst0, rows0, sem0)

            @pl.when(k1 < nci)
            def _():
                fire(k1 + 1, src0, dst0, rows0, sem0)
                drain_chunk(src1, dst1, rows1, sem1)

            return t

        lax.fori_loop(0, (nci + 1) // 2, body, 0)
        plsc.subcore_barrier()

        nk = n_dst // _CH
        nkt = (nk - s + _NSUB - 1) // _NSUB

        def drain(m, t):
            k = m * _NSUB + s
            pltpu.sync_copy(acc.at[pl.ds(k * _CH, _CH)], rows0.at[0, pl.ds(0, _CH)])
            pltpu.sync_copy(rows0.at[0, pl.ds(0, _CH)], out_hbm.at[c, pl.ds(k * _CH, _CH)])
            return t

        lax.fori_loop(0, nkt, drain, 0)

    return spmm


_B = 1000


def _inv_of(d_ref):
    deg = d_ref[0] + d_ref[1]
    return 1.0 / (jnp.sqrt(deg) + 1e-8)


def _scale_body(x_ref, d_ref, y_ref):
    inv = _inv_of(d_ref)
    y_ref[0] = x_ref[:, :_H] * inv
    y_ref[1] = x_ref[:, _H:] * inv


def _mid_body(s_ref, d_ref, z_ref, y_ref, *, denom):
    inv = _inv_of(d_ref)
    f0 = s_ref[0] * inv * (1.0 / denom)
    f1 = s_ref[1] * inv * (1.0 / denom)
    nrm2 = jnp.sum(f0 * f0, axis=1, keepdims=True) + jnp.sum(f1 * f1, axis=1, keepdims=True)
    scale = 1.0 / jnp.maximum(jnp.sqrt(nrm2), 1e-12)
    z_ref[...] = jnp.concatenate([f0, f1], axis=1) * scale
    y_ref[0] = f0 * inv
    y_ref[1] = f1 * inv


def _fin_body(s_ref, d_ref, x_ref, z1_ref, o_ref, *, denom):
    inv = _inv_of(d_ref)
    f0 = s_ref[0] * inv * (1.0 / denom)
    f1 = s_ref[1] * inv * (1.0 / denom)
    nrm2 = jnp.sum(f0 * f0, axis=1, keepdims=True) + jnp.sum(f1 * f1, axis=1, keepdims=True)
    scale = 1.0 / jnp.maximum(jnp.sqrt(nrm2), 1e-12)
    z2 = jnp.concatenate([f0, f1], axis=1) * scale
    o_ref[...] = x_ref[...] + z1_ref[...] + z2


def _fin_split_body(s_ref, d_ref, x_ref, z1_ref, o_ref, o2_ref, *, denom):
    inv = _inv_of(d_ref)
    f0 = s_ref[0] * inv * (1.0 / denom)
    f1 = s_ref[1] * inv * (1.0 / denom)
    nrm2 = jnp.sum(f0 * f0, axis=1, keepdims=True) + jnp.sum(f1 * f1, axis=1, keepdims=True)
    scale = 1.0 / jnp.maximum(jnp.sqrt(nrm2), 1e-12)
    z2 = jnp.concatenate([f0, f1], axis=1) * scale
    o = x_ref[...] + z1_ref[...] + z2
    o_ref[...] = o
    o2_ref[0] = o[:, :_H]
    o2_ref[1] = o[:, _H:]


def _bi_body(s_ref, d_ref, o_ref):
    cnt = d_ref[0] + d_ref[1]
    scale = 1.0 / (cnt + 1e-8)
    o_ref[...] = jnp.concatenate([s_ref[0], s_ref[1]], axis=1) * scale


def _spec_of(a):
    if a.ndim == 3 and a.shape[2] == 1:
        return pl.BlockSpec((2, _B, 1), lambda i: (0, i, 0))
    if a.ndim == 3:
        return pl.BlockSpec((2, _B, _H), lambda i: (0, i, 0))
    return pl.BlockSpec((_B, _D), lambda i: (i, 0))


def _tc_call(body, n, out_kinds, *args):
    shapes = {"full": jax.ShapeDtypeStruct((n, _D), jnp.float32),
              "split": jax.ShapeDtypeStruct((2, n, _H), jnp.float32)}
    specs = {"full": pl.BlockSpec((_B, _D), lambda i: (i, 0)),
             "split": pl.BlockSpec((2, _B, _H), lambda i: (0, i, 0))}
    res = pl.pallas_call(
        body,
        grid=(n // _B,),
        in_specs=[_spec_of(a) for a in args],
        out_specs=[specs[k] for k in out_kinds],
        out_shape=[shapes[k] for k in out_kinds],
    )(*args)
    return res


def _propagate(a_feat, b_feat, edge, n_a, n_b, split_tot_b=False):
    E = edge.shape[1]
    e0 = edge[0].astype(jnp.int32)
    e1 = edge[1].astype(jnp.int32)
    N = n_a + n_b
    all_idx = jnp.concatenate([e0, e1 + n_a]).reshape(-1, _K)
    degp = _make_deg(2 * E, N)(all_idx)
    e0 = e0.reshape(-1, _K)
    e1 = e1.reshape(-1, _K)
    da = degp[:, :n_a, :1]
    db = degp[:, n_a:N, :1]

    ya0 = _tc_call(_scale_body, n_a, ["split"], a_feat, da)[0]
    yb0 = _tc_call(_scale_body, n_b, ["split"], b_feat, db)[0]

    spmm_a = _make_spmm(E, n_a, n_b)
    spmm_b = _make_spmm(E, n_b, n_a)

    s1a = spmm_a(e0, e1, yb0.reshape(2 * n_b, _H))
    s1b = spmm_b(e1, e0, ya0.reshape(2 * n_a, _H))
    z1a, y1a = _tc_call(functools.partial(_mid_body, denom=2.0), n_a, ["full", "split"], s1a, da)
    z1b, y1b = _tc_call(functools.partial(_mid_body, denom=2.0), n_b, ["full", "split"], s1b, db)

    s2a = spmm_a(e0, e1, y1b.reshape(2 * n_b, _H))
    s2b = spmm_b(e1, e0, y1a.reshape(2 * n_a, _H))
    tot_a = _tc_call(functools.partial(_fin_body, denom=3.0), n_a, ["full"], s2a, da, a_feat, z1a)[0]
    if split_tot_b:
        tot_b, tot_b_split = _tc_call(
            functools.partial(_fin_split_body, denom=3.0), n_b, ["full", "split"], s2b, db, b_feat, z1b)
        return tot_a, tot_b, tot_b_split
    tot_b = _tc_call(functools.partial(_fin_body, denom=3.0), n_b, ["full"], s2b, db, b_feat, z1b)[0]
    return tot_a, tot_b


def kernel(users_feature, bundles_feature, items_feature, ui_edge_index, ub_edge_index, bi_edge_index):
    IL_users, IL_items, IL_items_split = _propagate(
        users_feature, items_feature, ui_edge_index, _NU, _NI, split_tot_b=True)
    BL_users, BL_bundles = _propagate(users_feature, bundles_feature, ub_edge_index, _NU, _NB)

    brow = bi_edge_index[0].astype(jnp.int32)
    bcol = bi_edge_index[1].astype(jnp.int32)
    Eb = brow.shape[0]
    sizep = _make_deg(Eb, _NB)(brow.reshape(-1, _K))
    dbi = sizep[:, :_NB, :1]
    sbi = _make_spmm(Eb, _NB, _NI)(
        brow.reshape(-1, _K), bcol.reshape(-1, _K), IL_items_split.reshape(2 * _NI, _H))
    IL_bundles = _tc_call(_bi_body, _NB, ["full"], sbi, dbi)[0]

    users_rep = jnp.concatenate([IL_users, BL_users], axis=1)
    bundles_rep = jnp.concatenate([IL_bundles, BL_bundles], axis=1)
    return jnp.concatenate([users_rep, bundles_rep], axis=0)
